# baseline (speedup 1.0000x reference)
import functools

import jax
import jax.numpy as jnp
from jax import lax
from jax.experimental import pallas as pl
from jax.experimental.pallas import tpu as pltpu
from jax.experimental.pallas import tpu_sc as plsc

B = 16384
NW = 32
PW = B // NW
CH = 128
NCH = PW // CH
L = 16
V = 1000000
BK = 131072
NBK = (V + BK - 1) // BK


def _split_gmf_body(gu_ref, gi_ref, o0_ref, o1_ref, o2_ref, o3_ref):
    o0_ref[...] = gu_ref[0, :]
    o1_ref[...] = gu_ref[1, :]
    o2_ref[...] = gi_ref[0, :]
    o3_ref[...] = gi_ref[1, :]


def _split_mlp_body(mu_ref, mi_ref, o4_ref, o5_ref):
    o4_ref[...] = mu_ref[0, :]
    o5_ref[...] = mi_ref[0, :]


def _split_gmf(gmf_u, gmf_i):
    row_spec = pl.BlockSpec((2, BK), lambda j: (0, j))
    col_spec = pl.BlockSpec((BK,), lambda j: (j,))
    return pl.pallas_call(
        _split_gmf_body,
        grid=(NBK,),
        in_specs=[row_spec, row_spec],
        out_specs=[col_spec] * 4,
        out_shape=[jax.ShapeDtypeStruct((V,), jnp.float32)] * 4,
    )(gmf_u.T, gmf_i.T)


def _split_mlp(mlp_u, mlp_i):
    one_spec = pl.BlockSpec((1, BK), lambda j: (0, j))
    col_spec = pl.BlockSpec((BK,), lambda j: (j,))
    return pl.pallas_call(
        _split_mlp_body,
        grid=(NBK,),
        in_specs=[one_spec, one_spec],
        out_specs=[col_spec] * 2,
        out_shape=[jax.ShapeDtypeStruct((V,), jnp.float32)] * 2,
    )(mlp_u.T, mlp_i.T)


def _gmf_gather_body(uu, ii, gu0, gu1, gi0, gi1, out_hbm,
                     idx_v, gat_v, sem):
    c = lax.axis_index("c")
    s = lax.axis_index("s")
    wid = s * 2 + c

    cps = [pltpu.async_copy(uu.at[wid], idx_v.at[0], sem),
           pltpu.async_copy(ii.at[wid], idx_v.at[1], sem)]
    for cp in cps:
        cp.wait()

    tabs = ((gu0, 0), (gu1, 0), (gi0, 1), (gi1, 1))
    gs = []
    for t, (tab, which) in enumerate(tabs):
        for j in range(NCH):
            gs.append(pltpu.async_copy(
                tab.at[idx_v.at[which, j]],
                gat_v.at[t, pl.ds(j * CH, CH)],
                sem))
    for g in gs:
        g.wait()

    pltpu.sync_copy(gat_v, out_hbm.at[wid])


def _mlp_final_body(uu, ii, mu, mi, gmf4, wmat, out_hbm,
                    idx_v, gat_v, g4_v, w_v, out_v, sem):
    c = lax.axis_index("c")
    s = lax.axis_index("s")
    wid = s * 2 + c

    cps = [pltpu.async_copy(uu.at[wid], idx_v.at[0], sem),
           pltpu.async_copy(ii.at[wid], idx_v.at[1], sem),
           pltpu.async_copy(gmf4.at[wid], g4_v, sem),
           pltpu.async_copy(wmat, w_v, sem)]
    for cp in cps:
        cp.wait()

    gs = []
    for t, (tab, which) in enumerate(((mu, 0), (mi, 1))):
        for j in range(NCH):
            gs.append(pltpu.async_copy(
                tab.at[idx_v.at[which, j]],
                gat_v.at[t, pl.ds(j * CH, CH)],
                sem))
    for g in gs:
        g.wait()

    w = [w_v[r] for r in range(29)]

    for i in range(PW // L):
        dv = pl.ds(i * L, L)
        a0 = g4_v[0, dv]
        a1 = g4_v[1, dv]
        b0 = g4_v[2, dv]
        b1 = g4_v[3, dv]
        x0 = gat_v[0, dv]
        x1 = gat_v[1, dv]
        g0 = a0 * b0
        g1 = a1 * b1
        for li in range(4):
            n0 = jnp.maximum(w[4 * li] * x0 + w[4 * li + 1] * x1
                             + w[16 + 2 * li], 0.0)
            n1 = jnp.maximum(w[4 * li + 2] * x0 + w[4 * li + 3] * x1
                             + w[16 + 2 * li + 1], 0.0)
            x0, x1 = n0, n1
        z = w[24] * g0 + w[25] * g1 + w[26] * x0 + w[27] * x1 + w[28]
        out_v[dv] = 1.0 / (1.0 + jnp.exp(-z))

    pltpu.sync_copy(out_v, out_hbm.at[wid])


_MESH = dict(core_axis_name="c", subcore_axis_name="s")


@jax.jit
def _ncf(uu, ii, gmf_u, gmf_i, mlp_u, mlp_i, wmat):
    gu0, gu1, gi0, gi1 = _split_gmf(gmf_u, gmf_i)
    mu, mi = _split_mlp(mlp_u, mlp_i)

    gmf4 = functools.partial(
        pl.kernel,
        out_type=jax.ShapeDtypeStruct((NW, 4, PW), jnp.float32),
        mesh=plsc.VectorSubcoreMesh(**_MESH),
        scratch_types=[
            pltpu.VMEM((2, NCH, CH), jnp.int32),
            pltpu.VMEM((4, PW), jnp.float32),
            pltpu.SemaphoreType.DMA,
        ],
    )(_gmf_gather_body)(uu, ii, gu0, gu1, gi0, gi1)

    out = functools.partial(
        pl.kernel,
        out_type=jax.ShapeDtypeStruct((NW, PW), jnp.float32),
        mesh=plsc.VectorSubcoreMesh(**_MESH),
        scratch_types=[
            pltpu.VMEM((2, NCH, CH), jnp.int32),
            pltpu.VMEM((2, PW), jnp.float32),
            pltpu.VMEM((4, PW), jnp.float32),
            pltpu.VMEM((29, L), jnp.float32),
            pltpu.VMEM((PW,), jnp.float32),
            pltpu.SemaphoreType.DMA,
        ],
    )(_mlp_final_body)(uu, ii, mu, mi, gmf4, wmat)
    return out


def kernel(user, item, gmf_user_w, gmf_item_w, mlp_user_w, mlp_item_w,
           fc_W, fc_b, pred_W, pred_b):
    shp = (NW, NCH, CH)
    uu = user.astype(jnp.int32).reshape(shp)
    ii = item.astype(jnp.int32).reshape(shp)
    w29 = jnp.concatenate([
        fc_W.reshape(-1),
        fc_b.reshape(-1),
        pred_W.reshape(-1),
        pred_b.reshape(-1),
    ])
    wmat = jnp.broadcast_to(w29[:, None], (29, L))
    out = _ncf(uu, ii, gmf_user_w, gmf_item_w, mlp_user_w, mlp_item_w, wmat)
    return out.reshape(B, 1)


# speedup vs baseline: 1.0399x; 1.0399x over previous
import functools

import jax
import jax.numpy as jnp
from jax import lax
from jax.experimental import pallas as pl
from jax.experimental.pallas import tpu as pltpu
from jax.experimental.pallas import tpu_sc as plsc

B = 16384
NW = 32
PW = B // NW
CH = 128
NCH = PW // CH
NT = 6
L = 16
V = 1000000
BK = 131072
NBK = (V + BK - 1) // BK


def _split_body(gu_ref, gi_ref, mu_ref, mi_ref,
                o0_ref, o1_ref, o2_ref, o3_ref, o4_ref, o5_ref):
    o0_ref[...] = gu_ref[0, :]
    o1_ref[...] = gu_ref[1, :]
    o2_ref[...] = gi_ref[0, :]
    o3_ref[...] = gi_ref[1, :]
    o4_ref[...] = mu_ref[0, :]
    o5_ref[...] = mi_ref[0, :]


def _split_columns(gmf_u, gmf_i, mlp_u, mlp_i):
    row_spec = pl.BlockSpec((2, BK), lambda j: (0, j))
    one_spec = pl.BlockSpec((1, BK), lambda j: (0, j))
    col_spec = pl.BlockSpec((BK,), lambda j: (j,))
    return pl.pallas_call(
        _split_body,
        grid=(NBK,),
        in_specs=[row_spec, row_spec, one_spec, one_spec],
        out_specs=[col_spec] * NT,
        out_shape=[jax.ShapeDtypeStruct((V,), jnp.float32)] * NT,
    )(gmf_u.T, gmf_i.T, mlp_u.T, mlp_i.T)


def _ncf_body(uu, ii, gu0, gu1, gi0, gi1, mu, mi, wmat,
              out_hbm,
              idx_v, gat_v, w_v, out_v, sem, csem):
    c = lax.axis_index("c")
    s = lax.axis_index("s")
    wid = s * 2 + c

    cps = [pltpu.async_copy(uu.at[wid], idx_v.at[0], sem),
           pltpu.async_copy(ii.at[wid], idx_v.at[1], sem),
           pltpu.async_copy(wmat, w_v, sem)]
    for cp in cps:
        cp.wait()

    tabs = ((gu0, 0), (gu1, 0), (gi0, 1), (gi1, 1), (mu, 0), (mi, 1))
    gs = {j: [] for j in range(NCH)}
    for t, (tab, which) in enumerate(tabs):
        for j in range(NCH):
            gs[j].append(pltpu.async_copy(
                tab.at[idx_v.at[which, j]],
                gat_v.at[t, pl.ds(j * CH, CH)],
                csem.at[j]))

    w = [w_v[r] for r in range(29)]

    for j in range(NCH):
        for g in gs[j]:
            g.wait()
        for ic in range(CH // L):
            i = j * (CH // L) + ic
            dv = pl.ds(i * L, L)
            a0 = gat_v[0, dv]
            a1 = gat_v[1, dv]
            b0 = gat_v[2, dv]
            b1 = gat_v[3, dv]
            x0 = gat_v[4, dv]
            x1 = gat_v[5, dv]
            g0 = a0 * b0
            g1 = a1 * b1
            for li in range(4):
                n0 = jnp.maximum(w[4 * li] * x0 + w[4 * li + 1] * x1
                                 + w[16 + 2 * li], 0.0)
                n1 = jnp.maximum(w[4 * li + 2] * x0 + w[4 * li + 3] * x1
                                 + w[16 + 2 * li + 1], 0.0)
                x0, x1 = n0, n1
            z = w[24] * g0 + w[25] * g1 + w[26] * x0 + w[27] * x1 + w[28]
            out_v[dv] = 1.0 / (1.0 + jnp.exp(-z))

    pltpu.sync_copy(out_v, out_hbm.at[wid])


@jax.jit
def _ncf_sc(uu, ii, gu0, gu1, gi0, gi1, mu, mi, wmat):
    mesh = plsc.VectorSubcoreMesh(core_axis_name="c", subcore_axis_name="s")
    run = functools.partial(
        pl.kernel,
        out_type=jax.ShapeDtypeStruct((NW, PW), jnp.float32),
        mesh=mesh,
        scratch_types=[
            pltpu.VMEM((2, NCH, CH), jnp.int32),
            pltpu.VMEM((NT, PW), jnp.float32),
            pltpu.VMEM((29, L), jnp.float32),
            pltpu.VMEM((PW,), jnp.float32),
            pltpu.SemaphoreType.DMA,
            pltpu.SemaphoreType.DMA((NCH,)),
        ],
    )(_ncf_body)
    return run(uu, ii, gu0, gu1, gi0, gi1, mu, mi, wmat)


def kernel(user, item, gmf_user_w, gmf_item_w, mlp_user_w, mlp_item_w,
           fc_W, fc_b, pred_W, pred_b):
    shp = (NW, NCH, CH)
    uu = user.astype(jnp.int32).reshape(shp)
    ii = item.astype(jnp.int32).reshape(shp)
    gu0, gu1, gi0, gi1, mu, mi = _split_columns(
        gmf_user_w, gmf_item_w, mlp_user_w, mlp_item_w)
    w29 = jnp.concatenate([
        fc_W.reshape(-1),
        fc_b.reshape(-1),
        pred_W.reshape(-1),
        pred_b.reshape(-1),
    ])
    wmat = jnp.broadcast_to(w29[:, None], (29, L))
    out = _ncf_sc(uu, ii, gu0, gu1, gi0, gi1, mu, mi, wmat)
    return out.reshape(B, 1)


# speedup vs baseline: 1.0430x; 1.0030x over previous
import functools

import jax
import jax.numpy as jnp
from jax import lax
from jax.experimental import pallas as pl
from jax.experimental.pallas import tpu as pltpu
from jax.experimental.pallas import tpu_sc as plsc

B = 16384
NW = 32
PW = B // NW
CH = 128
NCH = PW // CH
NT = 6
L = 16
V = 1000000
BK = 131072
NBK = (V + BK - 1) // BK


def _split_body(gu_ref, gi_ref, mu_ref, mi_ref,
                o0_ref, o1_ref, o2_ref, o3_ref, o4_ref, o5_ref):
    o0_ref[...] = gu_ref[0, :]
    o1_ref[...] = gu_ref[1, :]
    o2_ref[...] = gi_ref[0, :]
    o3_ref[...] = gi_ref[1, :]
    o4_ref[...] = mu_ref[0, :]
    o5_ref[...] = mi_ref[0, :]


def _split_columns(gmf_u, gmf_i, mlp_u, mlp_i):
    row_spec = pl.BlockSpec((2, BK), lambda j: (0, j))
    one_spec = pl.BlockSpec((1, BK), lambda j: (0, j))
    col_spec = pl.BlockSpec((BK,), lambda j: (j,))
    return pl.pallas_call(
        _split_body,
        grid=(NBK,),
        in_specs=[row_spec, row_spec, one_spec, one_spec],
        out_specs=[col_spec] * NT,
        out_shape=[jax.ShapeDtypeStruct((V,), jnp.float32)] * NT,
    )(gmf_u.T, gmf_i.T, mlp_u.T, mlp_i.T)


def _ncf_body(uu, ii, gu0, gu1, gi0, gi1, mu, mi, wmat,
              out_hbm,
              idx_v, gat_v, w_v, out_v, sem, csem):
    c = lax.axis_index("c")
    s = lax.axis_index("s")
    wid = s * 2 + c

    cps = [pltpu.async_copy(uu.at[wid], idx_v.at[0], sem),
           pltpu.async_copy(ii.at[wid], idx_v.at[1], sem),
           pltpu.async_copy(wmat, w_v, sem)]
    for cp in cps:
        cp.wait()

    tabs = ((gu0, 0), (gu1, 0), (gi0, 1), (gi1, 1), (mu, 0), (mi, 1))
    gs = {j: [] for j in range(NCH)}
    for t, (tab, which) in enumerate(tabs):
        for j in range(NCH):
            gs[j].append(pltpu.async_copy(
                tab.at[idx_v.at[which, j]],
                gat_v.at[t, pl.ds(j * CH, CH)],
                csem.at[j]))

    w = [w_v[r] for r in range(29)]

    for j in range(NCH):
        for g in gs[j]:
            g.wait()
        for ic in range(CH // L):
            i = j * (CH // L) + ic
            dv = pl.ds(i * L, L)
            a0 = gat_v[0, dv]
            a1 = gat_v[1, dv]
            b0 = gat_v[2, dv]
            b1 = gat_v[3, dv]
            x0 = gat_v[4, dv]
            x1 = gat_v[5, dv]
            g0 = a0 * b0
            g1 = a1 * b1
            for li in range(4):
                n0 = jnp.maximum(w[4 * li] * x0 + w[4 * li + 1] * x1
                                 + w[16 + 2 * li], 0.0)
                n1 = jnp.maximum(w[4 * li + 2] * x0 + w[4 * li + 3] * x1
                                 + w[16 + 2 * li + 1], 0.0)
                x0, x1 = n0, n1
            z = w[24] * g0 + w[25] * g1 + w[26] * x0 + w[27] * x1 + w[28]
            out_v[dv] = 1.0 / (1.0 + jnp.exp(-z))

    pltpu.sync_copy(out_v, out_hbm.at[wid])


@jax.jit
def _ncf_sc(uu, ii, gu0, gu1, gi0, gi1, mu, mi, wmat):
    mesh = plsc.VectorSubcoreMesh(core_axis_name="c", subcore_axis_name="s")
    run = functools.partial(
        pl.kernel,
        out_type=jax.ShapeDtypeStruct((NW, PW), jnp.float32),
        mesh=mesh,
        scratch_types=[
            pltpu.VMEM((2, NCH, CH), jnp.int32),
            pltpu.VMEM((NT, PW), jnp.float32),
            pltpu.VMEM((29, L), jnp.float32),
            pltpu.VMEM((PW,), jnp.float32),
            pltpu.SemaphoreType.DMA,
            pltpu.SemaphoreType.DMA((NCH,)),
        ],
        compiler_params=pltpu.CompilerParams(skip_device_barrier=True),
    )(_ncf_body)
    return run(uu, ii, gu0, gu1, gi0, gi1, mu, mi, wmat)


def kernel(user, item, gmf_user_w, gmf_item_w, mlp_user_w, mlp_item_w,
           fc_W, fc_b, pred_W, pred_b):
    shp = (NW, NCH, CH)
    uu = user.astype(jnp.int32).reshape(shp)
    ii = item.astype(jnp.int32).reshape(shp)
    gu0, gu1, gi0, gi1, mu, mi = _split_columns(
        gmf_user_w, gmf_item_w, mlp_user_w, mlp_item_w)
    w29 = jnp.concatenate([
        fc_W.reshape(-1),
        fc_b.reshape(-1),
        pred_W.reshape(-1),
        pred_b.reshape(-1),
    ])
    wmat = jnp.broadcast_to(w29[:, None], (29, L))
    out = _ncf_sc(uu, ii, gu0, gu1, gi0, gi1, mu, mi, wmat)
    return out.reshape(B, 1)


# speedup vs baseline: 1.1194x; 1.0732x over previous
import functools

import jax
import jax.numpy as jnp
from jax import lax
from jax.experimental import pallas as pl
from jax.experimental.pallas import tpu as pltpu
from jax.experimental.pallas import tpu_sc as plsc

B = 16384
NW = 32
PW = B // NW
CH = 128
NCH = PW // CH
NT = 6
L = 16
V = 1000000
BK = 262144
NBK = (V + BK - 1) // BK


def _split_body(gu_ref, gi_ref, mu_ref, mi_ref,
                o0_ref, o1_ref, o2_ref, o3_ref, o4_ref, o5_ref):
    o0_ref[...] = gu_ref[0, :]
    o1_ref[...] = gu_ref[1, :]
    o2_ref[...] = gi_ref[0, :]
    o3_ref[...] = gi_ref[1, :]
    o4_ref[...] = mu_ref[0, :]
    o5_ref[...] = mi_ref[0, :]


def _split_columns(gmf_u, gmf_i, mlp_u, mlp_i):
    row_spec = pl.BlockSpec((2, BK), lambda j: (0, j))
    one_spec = pl.BlockSpec((1, BK), lambda j: (0, j))
    col_spec = pl.BlockSpec((BK,), lambda j: (j,))
    return pl.pallas_call(
        _split_body,
        grid=(NBK,),
        in_specs=[row_spec, row_spec, one_spec, one_spec],
        out_specs=[col_spec] * NT,
        out_shape=[jax.ShapeDtypeStruct((V,), jnp.float32)] * NT,
    )(gmf_u.T, gmf_i.T, mlp_u.T, mlp_i.T)


def _ncf_body(user, item, gu0, gu1, gi0, gi1, mu, mi, wmat,
              out_hbm,
              idx_v, gat_v, w_v, out_v, sem, csem):
    c = lax.axis_index("c")
    s = lax.axis_index("s")
    wid = s * 2 + c
    base = wid * PW

    cps = [pltpu.async_copy(user.at[pl.ds(base, PW)], idx_v.at[0], sem),
           pltpu.async_copy(item.at[pl.ds(base, PW)], idx_v.at[1], sem),
           pltpu.async_copy(wmat, w_v, sem)]
    for cp in cps:
        cp.wait()

    tabs = ((gu0, 0), (gu1, 0), (gi0, 1), (gi1, 1), (mu, 0), (mi, 1))
    gs = {j: [] for j in range(NCH)}
    for t, (tab, which) in enumerate(tabs):
        for j in range(NCH):
            gs[j].append(pltpu.async_copy(
                tab.at[idx_v.at[which, pl.ds(j * CH, CH)]],
                gat_v.at[t, pl.ds(j * CH, CH)],
                csem.at[j]))

    w = [w_v[r] for r in range(29)]

    for j in range(NCH):
        for g in gs[j]:
            g.wait()
        for ic in range(CH // L):
            i = j * (CH // L) + ic
            dv = pl.ds(i * L, L)
            a0 = gat_v[0, dv]
            a1 = gat_v[1, dv]
            b0 = gat_v[2, dv]
            b1 = gat_v[3, dv]
            x0 = gat_v[4, dv]
            x1 = gat_v[5, dv]
            g0 = a0 * b0
            g1 = a1 * b1
            for li in range(4):
                n0 = jnp.maximum(w[4 * li] * x0 + w[4 * li + 1] * x1
                                 + w[16 + 2 * li], 0.0)
                n1 = jnp.maximum(w[4 * li + 2] * x0 + w[4 * li + 3] * x1
                                 + w[16 + 2 * li + 1], 0.0)
                x0, x1 = n0, n1
            z = w[24] * g0 + w[25] * g1 + w[26] * x0 + w[27] * x1 + w[28]
            out_v[dv] = 1.0 / (1.0 + jnp.exp(-z))

    pltpu.sync_copy(out_v, out_hbm.at[pl.ds(base, PW)])


@jax.jit
def _ncf_sc(user, item, gu0, gu1, gi0, gi1, mu, mi, wmat):
    mesh = plsc.VectorSubcoreMesh(core_axis_name="c", subcore_axis_name="s")
    run = functools.partial(
        pl.kernel,
        out_type=jax.ShapeDtypeStruct((B,), jnp.float32),
        mesh=mesh,
        scratch_types=[
            pltpu.VMEM((2, PW), jnp.int32),
            pltpu.VMEM((NT, PW), jnp.float32),
            pltpu.VMEM((29, L), jnp.float32),
            pltpu.VMEM((PW,), jnp.float32),
            pltpu.SemaphoreType.DMA,
            pltpu.SemaphoreType.DMA((NCH,)),
        ],
        compiler_params=pltpu.CompilerParams(skip_device_barrier=True),
    )(_ncf_body)
    return run(user, item, gu0, gu1, gi0, gi1, mu, mi, wmat)


def kernel(user, item, gmf_user_w, gmf_item_w, mlp_user_w, mlp_item_w,
           fc_W, fc_b, pred_W, pred_b):
    user = user.astype(jnp.int32)
    item = item.astype(jnp.int32)
    gu0, gu1, gi0, gi1, mu, mi = _split_columns(
        gmf_user_w, gmf_item_w, mlp_user_w, mlp_item_w)
    w29 = jnp.concatenate([
        fc_W.reshape(-1),
        fc_b.reshape(-1),
        pred_W.reshape(-1),
        pred_b.reshape(-1),
    ])
    wmat = jnp.broadcast_to(w29[:, None], (29, L))
    out = _ncf_sc(user, item, gu0, gu1, gi0, gi1, mu, mi, wmat)
    return out.reshape(B, 1)
